# baseline (speedup 1.0000x reference)
import functools

import jax
import jax.numpy as jnp
from jax import lax
from jax.experimental import pallas as pl
from jax.experimental.pallas import tpu as pltpu
from jax.experimental.pallas import tpu_sc as plsc

HIDDEN = 768
LANES = 16
HB = HIDDEN // LANES
NUM_CORES = 2
NUM_SUBCORES = 16
NUM_WORKERS = NUM_CORES * NUM_SUBCORES
CHUNK = 64
TC_BLK = 2048


def _sc_lookup(batch, seq):
    n_tokens = batch * seq
    tpw = n_tokens // NUM_WORKERS
    wps = seq // tpw
    n_chunks = tpw // CHUNK
    assert n_chunks % 2 == 0 and seq % tpw == 0

    mesh = plsc.VectorSubcoreMesh(core_axis_name="c", subcore_axis_name="s")

    @functools.partial(
        pl.kernel,
        out_type=jax.ShapeDtypeStruct((batch, seq, HIDDEN), jnp.float32),
        mesh=mesh,
        scratch_types=[
            pltpu.VMEM((5, HIDDEN), jnp.float32),
            pltpu.VMEM((tpw + LANES,), jnp.int32),
            pltpu.VMEM((tpw + LANES,), jnp.float32),
            pltpu.VMEM((2 * CHUNK, HIDDEN), jnp.float32),
            pltpu.SemaphoreType.DMA,
            pltpu.SemaphoreType.DMA,
        ],
    )
    def body(rp_hbm, mask_hbm, ktab_hbm, outk_hbm, tab, idx, msk, obuf,
             s0, s1):
        wid = lax.axis_index("s") * NUM_CORES + lax.axis_index("c")
        bi = wid // wps
        col = (wid % wps) * tpw

        pltpu.sync_copy(ktab_hbm, tab)
        pltpu.sync_copy(rp_hbm.at[bi, pl.ds(col, tpw)],
                        idx.at[pl.ds(0, tpw)])
        pltpu.sync_copy(mask_hbm.at[bi, pl.ds(col, tpw)],
                        msk.at[pl.ds(0, tpw)])

        def copy(par, row0):
            return pltpu.make_async_copy(
                obuf.at[pl.ds(par * CHUNK, CHUNK)],
                outk_hbm.at[bi, pl.ds(row0, CHUNK)], (s0, s1)[par])

        def chunk_body(c, carry):
            p = lax.rem(c, 2)

            @pl.when(c >= 2)
            def _():
                @pl.when(p == 0)
                def _():
                    copy(0, col).wait()

                @pl.when(p == 1)
                def _():
                    copy(1, col).wait()

            @plsc.parallel_loop(0, CHUNK, unroll=2)
            def tok_body(t):
                tok = c * CHUNK + t
                s = idx[pl.ds(tok, LANES)][0]
                m = msk[pl.ds(tok, LANES)][0]
                row = p * CHUNK + t
                for k in range(HB):
                    sl = pl.ds(k * LANES, LANES)
                    obuf[row, sl] = tab[s, sl] * m

            row0 = col + c * CHUNK

            @pl.when(p == 0)
            def _():
                copy(0, row0).start()

            @pl.when(p == 1)
            def _():
                copy(1, row0).start()

            return carry

        lax.fori_loop(0, n_chunks, chunk_body, 0, unroll=False)
        copy(0, col).wait()
        copy(1, col).wait()

    return body


def _tc_select_body(rp_ref, msk_ref, tab_ref, out_ref):
    rpv = rp_ref[0]
    m = msk_ref[0]
    tab = tab_ref[...]
    rows = lax.broadcasted_iota(jnp.int32, (5, TC_BLK), 0)
    eq = jnp.broadcast_to(rpv, (5, TC_BLK)) == rows
    onehot = jnp.where(eq, jnp.broadcast_to(m, (5, TC_BLK)), 0.0)
    out_ref[0] = lax.dot_general(
        onehot, tab, (((0,), (0,)), ((), ())),
        preferred_element_type=jnp.float32)


def _tc_lookup(batch, seq):
    n_blocks = batch * seq // TC_BLK
    return pl.pallas_call(
        _tc_select_body,
        grid=(n_blocks,),
        in_specs=[
            pl.BlockSpec((1, 1, TC_BLK), lambda i: (i, 0, 0)),
            pl.BlockSpec((1, 1, TC_BLK), lambda i: (i, 0, 0)),
            pl.BlockSpec((5, HIDDEN), lambda i: (0, 0)),
        ],
        out_specs=pl.BlockSpec((1, TC_BLK, HIDDEN), lambda i: (i, 0, 0)),
        out_shape=jax.ShapeDtypeStruct((n_blocks, TC_BLK, HIDDEN),
                                       jnp.float32),
    )


def kernel(relative_positions, entity_mask, entity_pos_key_table,
           entity_pos_value_table):
    b, s = relative_positions.shape
    rp = relative_positions.astype(jnp.int32)
    n_blocks = b * s // TC_BLK
    rp3 = rp.reshape(n_blocks, 1, TC_BLK)
    msk3 = entity_mask.reshape(n_blocks, 1, TC_BLK)

    out_k = _sc_lookup(b, s)(rp, entity_mask, entity_pos_key_table)
    out_v = _tc_lookup(b, s)(rp3, msk3, entity_pos_value_table)
    return out_k, out_v.reshape(b, s, HIDDEN)


# speedup vs baseline: 1.0172x; 1.0172x over previous
import functools

import jax
import jax.numpy as jnp
from jax import lax
from jax.experimental import pallas as pl
from jax.experimental.pallas import tpu as pltpu
from jax.experimental.pallas import tpu_sc as plsc

HIDDEN = 768
LANES = 16
HB = HIDDEN // LANES
NUM_CORES = 2
NUM_SUBCORES = 16
NUM_WORKERS = NUM_CORES * NUM_SUBCORES
CHUNK = 64
TC_BLK = 1024


def _sc_lookup(batch, seq):
    n_tokens = batch * seq
    tpw = n_tokens // NUM_WORKERS
    wps = seq // tpw
    n_chunks = tpw // CHUNK
    assert n_chunks % 2 == 0 and seq % tpw == 0

    mesh = plsc.VectorSubcoreMesh(core_axis_name="c", subcore_axis_name="s")

    @functools.partial(
        pl.kernel,
        out_type=jax.ShapeDtypeStruct((batch, seq, HIDDEN), jnp.float32),
        mesh=mesh,
        scratch_types=[
            pltpu.VMEM((5, HIDDEN), jnp.float32),
            pltpu.VMEM((tpw + LANES,), jnp.int32),
            pltpu.VMEM((tpw + LANES,), jnp.float32),
            pltpu.VMEM((2 * CHUNK, HIDDEN), jnp.float32),
            pltpu.SemaphoreType.DMA,
            pltpu.SemaphoreType.DMA,
        ],
    )
    def body(rp_hbm, mask_hbm, ktab_hbm, outk_hbm, tab, idx, msk, obuf,
             s0, s1):
        wid = lax.axis_index("s") * NUM_CORES + lax.axis_index("c")
        bi = wid // wps
        col = (wid % wps) * tpw

        pltpu.sync_copy(ktab_hbm, tab)
        pltpu.sync_copy(rp_hbm.at[bi, pl.ds(col, tpw)],
                        idx.at[pl.ds(0, tpw)])
        pltpu.sync_copy(mask_hbm.at[bi, pl.ds(col, tpw)],
                        msk.at[pl.ds(0, tpw)])

        def copy(par, row0):
            return pltpu.make_async_copy(
                obuf.at[pl.ds(par * CHUNK, CHUNK)],
                outk_hbm.at[bi, pl.ds(row0, CHUNK)], (s0, s1)[par])

        def chunk_body(c, carry):
            p = lax.rem(c, 2)

            @pl.when(c >= 2)
            def _():
                @pl.when(p == 0)
                def _():
                    copy(0, col).wait()

                @pl.when(p == 1)
                def _():
                    copy(1, col).wait()

            @plsc.parallel_loop(0, CHUNK, unroll=2)
            def tok_body(t):
                tok = c * CHUNK + t
                s = idx[pl.ds(tok, LANES)][0]
                m = msk[pl.ds(tok, LANES)][0]
                row = p * CHUNK + t
                for k in range(HB):
                    sl = pl.ds(k * LANES, LANES)
                    obuf[row, sl] = tab[s, sl] * m

            row0 = col + c * CHUNK

            @pl.when(p == 0)
            def _():
                copy(0, row0).start()

            @pl.when(p == 1)
            def _():
                copy(1, row0).start()

            return carry

        lax.fori_loop(0, n_chunks, chunk_body, 0, unroll=False)
        copy(0, col).wait()
        copy(1, col).wait()

    return body


def _tc_select_body(rp_ref, msk_ref, tab_ref, out_ref):
    rpv = rp_ref[0]
    m = msk_ref[0]
    tab = tab_ref[...]
    rows = lax.broadcasted_iota(jnp.int32, (5, TC_BLK), 0)
    eq = jnp.broadcast_to(rpv, (5, TC_BLK)) == rows
    onehot = jnp.where(eq, jnp.broadcast_to(m, (5, TC_BLK)), 0.0)
    out_ref[0] = lax.dot_general(
        onehot, tab, (((0,), (0,)), ((), ())),
        preferred_element_type=jnp.float32)


def _tc_lookup(batch, seq):
    n_blocks = batch * seq // TC_BLK
    return pl.pallas_call(
        _tc_select_body,
        grid=(n_blocks,),
        in_specs=[
            pl.BlockSpec((1, 1, TC_BLK), lambda i: (i, 0, 0)),
            pl.BlockSpec((1, 1, TC_BLK), lambda i: (i, 0, 0)),
            pl.BlockSpec((5, HIDDEN), lambda i: (0, 0)),
        ],
        out_specs=pl.BlockSpec((1, TC_BLK, HIDDEN), lambda i: (i, 0, 0)),
        out_shape=jax.ShapeDtypeStruct((n_blocks, TC_BLK, HIDDEN),
                                       jnp.float32),
    )


def kernel(relative_positions, entity_mask, entity_pos_key_table,
           entity_pos_value_table):
    b, s = relative_positions.shape
    rp = relative_positions.astype(jnp.int32)
    n_blocks = b * s // TC_BLK
    rp3 = rp.reshape(n_blocks, 1, TC_BLK)
    msk3 = entity_mask.reshape(n_blocks, 1, TC_BLK)

    out_k = _sc_lookup(b, s)(rp, entity_mask, entity_pos_key_table)
    out_v = _tc_lookup(b, s)(rp3, msk3, entity_pos_value_table)
    return out_k, out_v.reshape(b, s, HIDDEN)
